# Initial kernel scaffold; baseline (speedup 1.0000x reference)
import jax
import jax.numpy as jnp
from jax.experimental import pallas as pl


def kernel(rec_emb, rec_qkv, rec_proj, rec_fc1, rec_fc2, data_emb, data_qkv, data_proj, data_fc1, data_fc2):
    raise NotImplementedError("write your pallas kernel here")


# speedup vs baseline: 1.0050x; 1.0050x over previous
import jax
import jax.numpy as jnp
from jax.experimental import pallas as pl
from jax.experimental.pallas import tpu as pltpu

_NC = 2
_S = 24
_NB = _NC * _S

_EMB_ROWS = 50257
_EB = 1048

_SHAPES = {
    'emb':  (_EMB_ROWS, 768, _EB),
    'qkv':  (768, 2304, 16),
    'proj': (768, 768, 16),
    'fc1':  (3072, 768, 64),
    'fc2':  (768, 3072, 16),
}


def _tile_sum(x):
    b, w = x.shape
    if w != 768:
        parts = [x[:, i * 768:(i + 1) * 768] for i in range(w // 768)]
        x = parts[0]
        for p in parts[1:]:
            x = x + p
    return jnp.sum(x.reshape(b // 8, 8, 768), axis=0)


def _body(re_ref, rq_ref, rp_ref, rf1_ref, rf2_ref,
          de_ref, dq_ref, dp_ref, df1_ref, df2_ref,
          sp_ref, rn_ref, dn_ref):
    k = pl.program_id(0)

    @pl.when(k == 0)
    def _():
        sp_ref[...] = jnp.zeros_like(sp_ref)
        rn_ref[...] = jnp.zeros_like(rn_ref)
        dn_ref[...] = jnp.zeros_like(dn_ref)

    r = re_ref[...]
    d = de_ref[...]
    rows = jax.lax.broadcasted_iota(jnp.int32, (_EB, 768), 0)
    valid = rows < (_EMB_ROWS - k * _EB)
    r = jnp.where(valid, r, 0.0)
    d = jnp.where(valid, d, 0.0)
    sp = _tile_sum(r * d)
    rn = _tile_sum(r * r)
    dn = _tile_sum(d * d)

    for rr, dd in ((rq_ref, dq_ref), (rp_ref, dp_ref),
                   (rf1_ref, df1_ref), (rf2_ref, df2_ref)):
        r = rr[...]
        d = dd[...]
        sp = sp + _tile_sum(r * d)
        rn = rn + _tile_sum(r * r)
        dn = dn + _tile_sum(d * d)

    sp_ref[...] += sp
    rn_ref[...] += rn
    dn_ref[...] += dn


def _in_spec(name):
    _, cols, rb = _SHAPES[name]
    return pl.BlockSpec((rb, cols), lambda k: (k, 0))


def kernel(rec_emb, rec_qkv, rec_proj, rec_fc1, rec_fc2,
           data_emb, data_qkv, data_proj, data_fc1, data_fc2):
    out_specs = [pl.BlockSpec((8, 768), lambda k: (0, 0))] * 3
    out_shape = [jax.ShapeDtypeStruct((8, 768), jnp.float32)] * 3
    in_specs = [_in_spec(n) for n in ('emb', 'qkv', 'proj', 'fc1', 'fc2')] * 2

    sp, rn, dn = pl.pallas_call(
        _body,
        grid=(_NB,),
        in_specs=in_specs,
        out_specs=out_specs,
        out_shape=out_shape,
        compiler_params=pltpu.CompilerParams(
            dimension_semantics=("arbitrary",),
        ),
        name="cosine_objective",
    )(rec_emb, rec_qkv, rec_proj, rec_fc1, rec_fc2,
      data_emb, data_qkv, data_proj, data_fc1, data_fc2)

    sp = jnp.sum(sp)
    rn = jnp.sum(rn)
    dn = jnp.sum(dn)
    return 1.0 - sp / jnp.sqrt(rn) / jnp.sqrt(dn)


# speedup vs baseline: 1.0797x; 1.0743x over previous
import jax
import jax.numpy as jnp
from jax.experimental import pallas as pl
from jax.experimental.pallas import tpu as pltpu

_NB = 24

_EMB_ROWS = 50257
_EB = 2096

_SHAPES = {
    'emb':  (_EMB_ROWS, 768, _EB),
    'qkv':  (768, 2304, 32),
    'proj': (768, 768, 32),
    'fc1':  (3072, 768, 128),
    'fc2':  (768, 3072, 32),
}


def _tile_sum(x):
    b, w = x.shape
    if w != 768:
        parts = [x[:, i * 768:(i + 1) * 768] for i in range(w // 768)]
        x = parts[0]
        for p in parts[1:]:
            x = x + p
    return jnp.sum(x.reshape(b // 8, 8, 768), axis=0)


def _body(re_ref, rq_ref, rp_ref, rf1_ref, rf2_ref,
          de_ref, dq_ref, dp_ref, df1_ref, df2_ref,
          sp_ref, rn_ref, dn_ref):
    k = pl.program_id(0)

    @pl.when(k == 0)
    def _():
        sp_ref[...] = jnp.zeros_like(sp_ref)
        rn_ref[...] = jnp.zeros_like(rn_ref)
        dn_ref[...] = jnp.zeros_like(dn_ref)

    r = re_ref[...]
    d = de_ref[...]
    rows = jax.lax.broadcasted_iota(jnp.int32, (_EB, 768), 0)
    valid = rows < (_EMB_ROWS - k * _EB)
    r = jnp.where(valid, r, 0.0)
    d = jnp.where(valid, d, 0.0)
    sp = _tile_sum(r * d)
    rn = _tile_sum(r * r)
    dn = _tile_sum(d * d)

    for rr, dd in ((rq_ref, dq_ref), (rp_ref, dp_ref),
                   (rf1_ref, df1_ref), (rf2_ref, df2_ref)):
        r = rr[...]
        d = dd[...]
        sp = sp + _tile_sum(r * d)
        rn = rn + _tile_sum(r * r)
        dn = dn + _tile_sum(d * d)

    sp_ref[...] += sp
    rn_ref[...] += rn
    dn_ref[...] += dn


def _in_spec(name):
    _, cols, rb = _SHAPES[name]
    return pl.BlockSpec((rb, cols), lambda k: (k, 0))


def kernel(rec_emb, rec_qkv, rec_proj, rec_fc1, rec_fc2,
           data_emb, data_qkv, data_proj, data_fc1, data_fc2):
    out_specs = [pl.BlockSpec((8, 768), lambda k: (0, 0))] * 3
    out_shape = [jax.ShapeDtypeStruct((8, 768), jnp.float32)] * 3
    in_specs = [_in_spec(n) for n in ('emb', 'qkv', 'proj', 'fc1', 'fc2')] * 2

    sp, rn, dn = pl.pallas_call(
        _body,
        grid=(_NB,),
        in_specs=in_specs,
        out_specs=out_specs,
        out_shape=out_shape,
        compiler_params=pltpu.CompilerParams(
            dimension_semantics=("arbitrary",),
            vmem_limit_bytes=56 * 1024 * 1024,
        ),
        name="cosine_objective",
    )(rec_emb, rec_qkv, rec_proj, rec_fc1, rec_fc2,
      data_emb, data_qkv, data_proj, data_fc1, data_fc2)

    sp = jnp.sum(sp)
    rn = jnp.sum(rn)
    dn = jnp.sum(dn)
    return 1.0 - sp / jnp.sqrt(rn) / jnp.sqrt(dn)
